# baseline (speedup 1.0000x reference)
import functools

import jax
import jax.numpy as jnp
from jax import lax
from jax.experimental import pallas as pl
from jax.experimental.pallas import tpu as pltpu
from jax.experimental.pallas import tpu_sc as plsc

D = 64
LANES = 16
NC, NS = 2, 16
NW = NC * NS
B = 16384
BW = B // NW
CHUNK = 128
NCH = BW // CHUNK
NPASS = 2
PW = BW // NPASS
CPP = NCH // NPASS

_mesh = plsc.VectorSubcoreMesh(core_axis_name="c", subcore_axis_name="s")

_cp = pltpu.CompilerParams(needs_layout_passes=False)


@functools.partial(
    pl.kernel,
    compiler_params=_cp,
    out_type=(
        jax.ShapeDtypeStruct((B,), jnp.float32),
        jax.ShapeDtypeStruct((B,), jnp.float32),
    ),
    mesh=_mesh,
    scratch_types=[
        pltpu.VMEM((NCH, CHUNK), jnp.int32),
        pltpu.VMEM((NCH, CHUNK), jnp.int32),
        pltpu.VMEM((NCH, CHUNK), jnp.int32),
        pltpu.VMEM((BW,), jnp.int32),
        pltpu.VMEM((BW,), jnp.int32),
        pltpu.VMEM((BW,), jnp.int32),
        pltpu.VMEM((PW, 2 * D), jnp.float32),
        pltpu.VMEM((PW, 2 * D), jnp.float32),
        pltpu.VMEM((PW, 2 * D), jnp.float32),
        pltpu.VMEM((BW,), jnp.float32),
        pltpu.VMEM((BW,), jnp.float32),
        pltpu.SemaphoreType.DMA,
    ],
)
def _bpr_sc(user_table_hbm, item_table_hbm, urows_hbm, irows_hbm, jrows_hbm,
            uhalf_hbm, ihalf_hbm, jhalf_hbm,
            out_i_hbm, out_j_hbm,
            idx_u, idx_i, idx_j, half_u, half_i, half_j,
            u_rows, i_rows, j_rows, oi, oj, sem):
    wid = lax.axis_index("s") * NC + lax.axis_index("c")
    base = wid * BW

    pltpu.sync_copy(urows_hbm.at[wid], idx_u)
    pltpu.sync_copy(irows_hbm.at[wid], idx_i)
    pltpu.sync_copy(jrows_hbm.at[wid], idx_j)
    pltpu.sync_copy(uhalf_hbm.at[wid], half_u)
    pltpu.sync_copy(ihalf_hbm.at[wid], half_i)
    pltpu.sync_copy(jhalf_hbm.at[wid], half_j)

    lane = lax.iota(jnp.int32, LANES)

    for p in range(NPASS):
        copies = []
        for c in range(CPP):
            ch = p * CPP + c
            rows = pl.ds(c * CHUNK, CHUNK)
            copies.append(pltpu.async_copy(
                user_table_hbm.at[idx_u.at[ch]], u_rows.at[rows], sem))
            copies.append(pltpu.async_copy(
                item_table_hbm.at[idx_i.at[ch]], i_rows.at[rows], sem))
            copies.append(pltpu.async_copy(
                item_table_hbm.at[idx_j.at[ch]], j_rows.at[rows], sem))
        for cp in copies:
            cp.wait()

        @pl.loop(0, PW, step=LANES)
        def _(r0):
            row = r0 + lane
            col_u = half_u[pl.ds(p * PW + r0, LANES)]
            col_i = half_i[pl.ds(p * PW + r0, LANES)]
            col_j = half_j[pl.ds(p * PW + r0, LANES)]
            acc_i = jnp.zeros((LANES,), jnp.float32)
            acc_j = jnp.zeros((LANES,), jnp.float32)
            for d in range(D):
                u = plsc.load_gather(u_rows, [row, col_u + d])
                vi = plsc.load_gather(i_rows, [row, col_i + d])
                vj = plsc.load_gather(j_rows, [row, col_j + d])
                acc_i = acc_i + u * vi
                acc_j = acc_j + u * vj
            oi[pl.ds(p * PW + r0, LANES)] = acc_i
            oj[pl.ds(p * PW + r0, LANES)] = acc_j

    pltpu.sync_copy(oi, out_i_hbm.at[pl.ds(base, BW)])
    pltpu.sync_copy(oj, out_j_hbm.at[pl.ds(base, BW)])


def kernel(user_table, item_table, user, item_i, item_j):
    ut = user_table.reshape(user_table.shape[0] // 2, 2 * D)
    it = item_table.reshape(item_table.shape[0] // 2, 2 * D)
    user = user.astype(jnp.int32)
    item_i = item_i.astype(jnp.int32)
    item_j = item_j.astype(jnp.int32)
    urows = (user >> 1).reshape(NW, NCH, CHUNK)
    irows = (item_i >> 1).reshape(NW, NCH, CHUNK)
    jrows = (item_j >> 1).reshape(NW, NCH, CHUNK)
    uhalf = ((user & 1) * D).reshape(NW, BW)
    ihalf = ((item_i & 1) * D).reshape(NW, BW)
    jhalf = ((item_j & 1) * D).reshape(NW, BW)
    return _bpr_sc(ut, it, urows, irows, jrows, uhalf, ihalf, jhalf)


# speedup vs baseline: 1.0023x; 1.0023x over previous
import functools

import jax
import jax.numpy as jnp
from jax import lax
from jax.experimental import pallas as pl
from jax.experimental.pallas import tpu as pltpu
from jax.experimental.pallas import tpu_sc as plsc

D = 64
LANES = 16
NC, NS = 2, 16
NW = NC * NS
B = 16384
BW = B // NW
CHUNK = 128
NCH = BW // CHUNK
NPASS = 2
PW = BW // NPASS
CPP = NCH // NPASS

_mesh = plsc.VectorSubcoreMesh(core_axis_name="c", subcore_axis_name="s")

_cp = pltpu.CompilerParams(
    needs_layout_passes=False,
    use_tc_tiling_on_sc=False,
)


@functools.partial(
    pl.kernel,
    compiler_params=_cp,
    out_type=(
        jax.ShapeDtypeStruct((B,), jnp.float32),
        jax.ShapeDtypeStruct((B,), jnp.float32),
    ),
    mesh=_mesh,
    scratch_types=[
        pltpu.VMEM((NCH, CHUNK), jnp.int32),
        pltpu.VMEM((NCH, CHUNK), jnp.int32),
        pltpu.VMEM((NCH, CHUNK), jnp.int32),
        pltpu.VMEM((BW,), jnp.int32),
        pltpu.VMEM((BW,), jnp.int32),
        pltpu.VMEM((BW,), jnp.int32),
        pltpu.VMEM((PW, 2 * D), jnp.float32),
        pltpu.VMEM((PW, 2 * D), jnp.float32),
        pltpu.VMEM((PW, 2 * D), jnp.float32),
        pltpu.VMEM((BW,), jnp.float32),
        pltpu.VMEM((BW,), jnp.float32),
        pltpu.SemaphoreType.DMA,
    ],
)
def _bpr_sc(user_table_hbm, item_table_hbm, urows_hbm, irows_hbm, jrows_hbm,
            uhalf_hbm, ihalf_hbm, jhalf_hbm,
            out_i_hbm, out_j_hbm,
            idx_u, idx_i, idx_j, half_u, half_i, half_j,
            u_rows, i_rows, j_rows, oi, oj, sem):
    wid = lax.axis_index("s") * NC + lax.axis_index("c")
    base = wid * BW

    pltpu.sync_copy(urows_hbm.at[wid], idx_u)
    pltpu.sync_copy(irows_hbm.at[wid], idx_i)
    pltpu.sync_copy(jrows_hbm.at[wid], idx_j)
    pltpu.sync_copy(uhalf_hbm.at[wid], half_u)
    pltpu.sync_copy(ihalf_hbm.at[wid], half_i)
    pltpu.sync_copy(jhalf_hbm.at[wid], half_j)

    lane = lax.iota(jnp.int32, LANES)

    for p in range(NPASS):
        copies = []
        for c in range(CPP):
            ch = p * CPP + c
            rows = pl.ds(c * CHUNK, CHUNK)
            copies.append(pltpu.async_copy(
                user_table_hbm.at[idx_u.at[ch]], u_rows.at[rows], sem))
            copies.append(pltpu.async_copy(
                item_table_hbm.at[idx_i.at[ch]], i_rows.at[rows], sem))
            copies.append(pltpu.async_copy(
                item_table_hbm.at[idx_j.at[ch]], j_rows.at[rows], sem))
        for cp in copies:
            cp.wait()

        @pl.loop(0, PW, step=LANES)
        def _(r0):
            row = r0 + lane
            col_u = half_u[pl.ds(p * PW + r0, LANES)]
            col_i = half_i[pl.ds(p * PW + r0, LANES)]
            col_j = half_j[pl.ds(p * PW + r0, LANES)]
            acc_i = jnp.zeros((LANES,), jnp.float32)
            acc_j = jnp.zeros((LANES,), jnp.float32)
            for d in range(D):
                u = plsc.load_gather(u_rows, [row, col_u + d])
                vi = plsc.load_gather(i_rows, [row, col_i + d])
                vj = plsc.load_gather(j_rows, [row, col_j + d])
                acc_i = acc_i + u * vi
                acc_j = acc_j + u * vj
            oi[pl.ds(p * PW + r0, LANES)] = acc_i
            oj[pl.ds(p * PW + r0, LANES)] = acc_j

    pltpu.sync_copy(oi, out_i_hbm.at[pl.ds(base, BW)])
    pltpu.sync_copy(oj, out_j_hbm.at[pl.ds(base, BW)])


def kernel(user_table, item_table, user, item_i, item_j):
    user = user.astype(jnp.int32)
    item_i = item_i.astype(jnp.int32)
    item_j = item_j.astype(jnp.int32)
    ut = user_table.reshape(user_table.shape[0] // 2, 2 * D)
    it = item_table.reshape(item_table.shape[0] // 2, 2 * D)
    urows = (user >> 1).reshape(NW, NCH, CHUNK)
    irows = (item_i >> 1).reshape(NW, NCH, CHUNK)
    jrows = (item_j >> 1).reshape(NW, NCH, CHUNK)
    uhalf = ((user & 1) * D).reshape(NW, BW)
    ihalf = ((item_i & 1) * D).reshape(NW, BW)
    jhalf = ((item_j & 1) * D).reshape(NW, BW)
    return _bpr_sc(ut, it, urows, irows, jrows, uhalf, ihalf, jhalf)


# speedup vs baseline: 1.0409x; 1.0385x over previous
import functools

import jax
import jax.numpy as jnp
from jax import lax
from jax.experimental import pallas as pl
from jax.experimental.pallas import tpu as pltpu
from jax.experimental.pallas import tpu_sc as plsc

D = 64
LANES = 16
NC, NS = 2, 16
NW = NC * NS
B = 16384
BW = B // NW
CHUNK = 128
NCH = BW // CHUNK

_mesh = plsc.VectorSubcoreMesh(core_axis_name="c", subcore_axis_name="s")

_cp = pltpu.CompilerParams(
    needs_layout_passes=False,
    use_tc_tiling_on_sc=False,
)


@functools.partial(
    pl.kernel,
    compiler_params=_cp,
    out_type=(
        jax.ShapeDtypeStruct((B,), jnp.float32),
        jax.ShapeDtypeStruct((B,), jnp.float32),
    ),
    mesh=_mesh,
    scratch_types=[
        pltpu.VMEM((NCH, CHUNK), jnp.int32),
        pltpu.VMEM((NCH, CHUNK), jnp.int32),
        pltpu.VMEM((NCH, CHUNK), jnp.int32),
        pltpu.VMEM((BW, D), jnp.float32),
        pltpu.VMEM((BW, D), jnp.float32),
        pltpu.VMEM((BW, D), jnp.float32),
        pltpu.VMEM((BW,), jnp.float32),
        pltpu.VMEM((BW,), jnp.float32),
        pltpu.SemaphoreType.DMA,
    ],
)
def _bpr_sc(user_table_hbm, item_table_hbm, user_hbm, item_i_hbm, item_j_hbm,
            out_i_hbm, out_j_hbm,
            idx_u, idx_i, idx_j, u_rows, i_rows, j_rows, oi, oj, sem):
    wid = lax.axis_index("s") * NC + lax.axis_index("c")
    base = wid * BW

    pltpu.sync_copy(user_hbm.at[wid], idx_u)
    pltpu.sync_copy(item_i_hbm.at[wid], idx_i)
    pltpu.sync_copy(item_j_hbm.at[wid], idx_j)

    copies = []
    for c in range(NCH):
        rows = pl.ds(c * CHUNK, CHUNK)
        copies.append(pltpu.async_copy(
            user_table_hbm.at[idx_u.at[c]], u_rows.at[rows], sem))
        copies.append(pltpu.async_copy(
            item_table_hbm.at[idx_i.at[c]], i_rows.at[rows], sem))
        copies.append(pltpu.async_copy(
            item_table_hbm.at[idx_j.at[c]], j_rows.at[rows], sem))
    for cp in copies:
        cp.wait()

    lane = lax.iota(jnp.int32, LANES)

    @pl.loop(0, BW, step=LANES)
    def _(r0):
        res_i = jnp.zeros((LANES,), jnp.float32)
        res_j = jnp.zeros((LANES,), jnp.float32)
        for rr in range(LANES):
            r = r0 + rr
            acc_i = jnp.zeros((LANES,), jnp.float32)
            acc_j = jnp.zeros((LANES,), jnp.float32)
            for c in range(D // LANES):
                cols = pl.ds(c * LANES, LANES)
                u = u_rows[r, cols]
                acc_i = acc_i + u * i_rows[r, cols]
                acc_j = acc_j + u * j_rows[r, cols]
            res_i = jnp.where(lane == rr, jnp.sum(acc_i), res_i)
            res_j = jnp.where(lane == rr, jnp.sum(acc_j), res_j)
        oi[pl.ds(r0, LANES)] = res_i
        oj[pl.ds(r0, LANES)] = res_j

    pltpu.sync_copy(oi, out_i_hbm.at[pl.ds(base, BW)])
    pltpu.sync_copy(oj, out_j_hbm.at[pl.ds(base, BW)])


def kernel(user_table, item_table, user, item_i, item_j):
    u = user.astype(jnp.int32).reshape(NW, NCH, CHUNK)
    ii = item_i.astype(jnp.int32).reshape(NW, NCH, CHUNK)
    ij = item_j.astype(jnp.int32).reshape(NW, NCH, CHUNK)
    return _bpr_sc(user_table, item_table, u, ii, ij)
